# baseline (speedup 1.0000x reference)
import functools

import jax
import jax.numpy as jnp
from jax import lax
from jax.experimental import pallas as pl
from jax.experimental.pallas import tpu as pltpu
from jax.experimental.pallas import tpu_sc as plsc

N = 10000
NP = 10240
E = 320000
EP = 327680
D = 128
CD = 2 * D
VOC = 20000
T = 16
G = 256
GPOOL = 384
NC = 2
NS = 16
K = 80
RPT = NP // (NC * NS)
EPC = EP // NC
EPT_DEG = EPC // NS
EPT = EP // NS
NPT = NP // NS
GPT = GPOOL // NS
EK = 128
NSUB = 8
SK = NSUB * EK

_MESH = plsc.VectorSubcoreMesh(core_axis_name="c", subcore_axis_name="s")


def _fill1d(ref, n, val):
    def body(i, _):
        ref[pl.ds(i * 16, 16)] = jnp.full((16,), val, jnp.float32)
        return 0
    lax.fori_loop(0, n // 16, body, 0)


@functools.partial(
    pl.kernel,
    mesh=_MESH,
    out_type=[
        jax.ShapeDtypeStruct((NP, D), jnp.float32),
        jax.ShapeDtypeStruct((NP, 768), jnp.float32),
        jax.ShapeDtypeStruct((NC, NP), jnp.float32),
    ],
    scratch_types=[
        pltpu.VMEM((K,), jnp.int32),
        pltpu.VMEM((NSUB, EK), jnp.int32),
        pltpu.VMEM((K, D), jnp.float32),
        pltpu.VMEM((K, 768), jnp.float32),
        pltpu.VMEM((EK,), jnp.float32),
        pltpu.VMEM((NPT,), jnp.float32),
        pltpu.SemaphoreType.DMA,
        pltpu.SemaphoreType.DMA,
        pltpu.VMEM_SHARED((NP,), jnp.float32),
    ],
)
def _encode_call(x_hbm, dst3_hbm, onto_hbm, text_hbm,
                 onto_out, text_out, deg_out,
                 idx_v, didx_v, onto_v, text_v, ones_v, nbuf_v, sem, sem2,
                 deg_sp):
    c = lax.axis_index("c")
    s = lax.axis_index("s")
    wid = s * NC + c

    _fill1d(ones_v, EK, 1.0)
    _fill1d(nbuf_v, NPT, 0.0)
    pltpu.sync_copy(nbuf_v, deg_sp.at[pl.ds(s * NPT, NPT)])
    plsc.subcore_barrier()

    def gbody(j, _):
        base = wid * RPT + j * K
        pltpu.sync_copy(x_hbm.at[pl.ds(base, K)], idx_v)
        cp1 = pltpu.async_copy(onto_hbm.at[idx_v], onto_v, sem)
        cp2 = pltpu.async_copy(text_hbm.at[idx_v], text_v, sem2)
        cp1.wait()
        co1 = pltpu.async_copy(onto_v, onto_out.at[pl.ds(base, K)], sem)
        cp2.wait()
        co2 = pltpu.async_copy(text_v, text_out.at[pl.ds(base, K)], sem2)
        co1.wait()
        co2.wait()
        return 0
    lax.fori_loop(0, RPT // K, gbody, 0)

    def dbody(j, _):
        rowb = pl.multiple_of((c * EPC + s * EPT_DEG) // EK + j * NSUB, 8)
        pltpu.sync_copy(dst3_hbm.at[pl.ds(rowb, NSUB)], didx_v)
        cps = [pltpu.async_copy(ones_v, deg_sp.at[didx_v.at[jj]], sem,
                                add=True)
               for jj in range(NSUB)]
        for cp in cps:
            cp.wait()
        return 0
    lax.fori_loop(0, EPT_DEG // SK, dbody, 0)
    plsc.subcore_barrier()

    pltpu.sync_copy(deg_sp.at[pl.ds(s * NPT, NPT)], nbuf_v)
    pltpu.sync_copy(nbuf_v, deg_out.at[c, pl.ds(s * NPT, NPT)])


@functools.partial(
    pl.kernel,
    mesh=_MESH,
    out_type=[
        jax.ShapeDtypeStruct((NP, D), jnp.float32),
        jax.ShapeDtypeStruct((NP, D), jnp.float32),
    ],
    scratch_types=[
        pltpu.VMEM((K,), jnp.int32),
        pltpu.VMEM((K, D), jnp.float32),
        pltpu.SemaphoreType.DMA,
        pltpu.VMEM_SHARED((NP, D), jnp.float32),
    ],
)
def _agg_call(ma_hbm, mb_hbm, src_hbm, dst_hbm,
              oa_hbm, ob_hbm,
              idx_v, rows_v, sem, acc_sp):
    c = lax.axis_index("c")
    s = lax.axis_index("s")

    def run(m_hbm, o_hbm):
        def ibody(j, _):
            base = s * NPT + j * K
            pltpu.sync_copy(m_hbm.at[pl.ds(base, K)], rows_v)
            pltpu.sync_copy(rows_v, acc_sp.at[pl.ds(base, K)])
            return 0
        lax.fori_loop(0, NPT // K, ibody, 0)
        plsc.subcore_barrier()

        def ebody(j, _):
            eb = s * EPT + j * K
            pltpu.sync_copy(src_hbm.at[pl.ds(eb, K)], idx_v)
            pltpu.async_copy(m_hbm.at[idx_v], rows_v, sem).wait()
            pltpu.sync_copy(dst_hbm.at[pl.ds(eb, K)], idx_v)
            pltpu.sync_copy(rows_v, acc_sp.at[idx_v], add=True)
            return 0
        lax.fori_loop(0, EPT // K, ebody, 0)
        plsc.subcore_barrier()

        def obody(j, _):
            base = s * NPT + j * K
            pltpu.sync_copy(acc_sp.at[pl.ds(base, K)], rows_v)
            pltpu.sync_copy(rows_v, o_hbm.at[pl.ds(base, K)])
            return 0
        lax.fori_loop(0, NPT // K, obody, 0)

    pl.when(c == 0)(lambda: run(ma_hbm, oa_hbm))
    pl.when(c == 1)(lambda: run(mb_hbm, ob_hbm))


@functools.partial(
    pl.kernel,
    mesh=_MESH,
    out_type=[
        jax.ShapeDtypeStruct((GPOOL, D), jnp.float32),
        jax.ShapeDtypeStruct((GPOOL, D), jnp.float32),
        jax.ShapeDtypeStruct((GPOOL, D), jnp.float32),
    ],
    scratch_types=[
        pltpu.VMEM((K,), jnp.int32),
        pltpu.VMEM((K, D), jnp.float32),
        pltpu.VMEM((K, D), jnp.float32),
        pltpu.SemaphoreType.DMA,
        pltpu.VMEM_SHARED((GPOOL, D), jnp.float32),
        pltpu.VMEM_SHARED((GPOOL, D), jnp.float32),
    ],
)
def _pool_call(ha_hbm, hb_hbm, xb_hbm,
               sa_hbm, sb_hbm, cnt_hbm,
               idx_v, buf_v, ones_v, sem, pool_sp, cnt_sp):
    c = lax.axis_index("c")
    s = lax.axis_index("s")

    def zrow(i, _):
        for q in range(D // 16):
            buf_v[i, pl.ds(q * 16, 16)] = jnp.zeros((16,), jnp.float32)
            ones_v[i, pl.ds(q * 16, 16)] = jnp.full((16,), 1.0, jnp.float32)
        return 0
    lax.fori_loop(0, K, zrow, 0)

    pltpu.sync_copy(buf_v.at[pl.ds(0, GPT)], pool_sp.at[pl.ds(s * GPT, GPT)])
    pltpu.sync_copy(buf_v.at[pl.ds(0, GPT)], cnt_sp.at[pl.ds(s * GPT, GPT)])
    plsc.subcore_barrier()

    def run(h_hbm):
        def pbody(j, _):
            base = s * NPT + j * K
            pltpu.sync_copy(xb_hbm.at[pl.ds(base, K)], idx_v)
            pltpu.sync_copy(h_hbm.at[pl.ds(base, K)], buf_v)
            pltpu.sync_copy(buf_v, pool_sp.at[idx_v], add=True)

            @pl.when(c == 0)
            def _():
                pltpu.sync_copy(ones_v, cnt_sp.at[idx_v], add=True)
            return 0
        lax.fori_loop(0, NPT // K, pbody, 0)

    pl.when(c == 0)(lambda: run(ha_hbm))
    pl.when(c == 1)(lambda: run(hb_hbm))
    plsc.subcore_barrier()

    pltpu.sync_copy(pool_sp.at[pl.ds(s * GPT, GPT)], buf_v.at[pl.ds(0, GPT)])

    @pl.when(c == 0)
    def _():
        pltpu.sync_copy(buf_v.at[pl.ds(0, GPT)], sa_hbm.at[pl.ds(s * GPT, GPT)])
        pltpu.sync_copy(cnt_sp.at[pl.ds(s * GPT, GPT)], ones_v.at[pl.ds(0, GPT)])
        pltpu.sync_copy(ones_v.at[pl.ds(0, GPT)], cnt_hbm.at[pl.ds(s * GPT, GPT)])

    @pl.when(c == 1)
    def _():
        pltpu.sync_copy(buf_v.at[pl.ds(0, GPT)], sb_hbm.at[pl.ds(s * GPT, GPT)])


def _tca_body(onto_ref, text_ref, degT_ref, wt_ref, bt_ref,
              ma_ref, mb_ref, dv_ref):
    dv = lax.rsqrt(degT_ref[:, 0:1] + degT_ref[:, 1:2] + 1.0)
    t = jnp.dot(text_ref[...], wt_ref[...],
                preferred_element_type=jnp.float32) + bt_ref[...]
    ma_ref[...] = onto_ref[...] * dv
    mb_ref[...] = t * dv
    dv_ref[...] = dv


_tc_a = pl.pallas_call(
    _tca_body,
    grid=(NP // 256,),
    in_specs=[
        pl.BlockSpec((256, D), lambda i: (i, 0)),
        pl.BlockSpec((256, 768), lambda i: (i, 0)),
        pl.BlockSpec((256, NC), lambda i: (i, 0)),
        pl.BlockSpec((768, D), lambda i: (0, 0)),
        pl.BlockSpec((1, D), lambda i: (0, 0)),
    ],
    out_specs=[
        pl.BlockSpec((256, D), lambda i: (i, 0)),
        pl.BlockSpec((256, D), lambda i: (i, 0)),
        pl.BlockSpec((256, 1), lambda i: (i, 0)),
    ],
    out_shape=[
        jax.ShapeDtypeStruct((NP, D), jnp.float32),
        jax.ShapeDtypeStruct((NP, D), jnp.float32),
        jax.ShapeDtypeStruct((NP, 1), jnp.float32),
    ],
)


def _tcb_body(aa_ref, ab_ref, dv_ref, wg1_ref, m2a_ref, m2b_ref):
    dv = dv_ref[...]
    u = jnp.concatenate([aa_ref[...], ab_ref[...]], axis=1) * dv
    h1 = jnp.maximum(jnp.dot(u, wg1_ref[...],
                             preferred_element_type=jnp.float32), 0.0)
    m2 = h1 * dv
    m2a_ref[...] = m2[:, 0:D]
    m2b_ref[...] = m2[:, D:CD]


def _tcc_body(aa_ref, ab_ref, dv_ref, wg2_ref, h2a_ref, h2b_ref):
    u = jnp.concatenate([aa_ref[...], ab_ref[...]], axis=1) * dv_ref[...]
    h2 = jnp.dot(u, wg2_ref[...], preferred_element_type=jnp.float32)
    h2a_ref[...] = h2[:, 0:D]
    h2b_ref[...] = h2[:, D:CD]


_tc_b = pl.pallas_call(
    _tcb_body,
    grid=(NP // 256,),
    in_specs=[
        pl.BlockSpec((256, D), lambda i: (i, 0)),
        pl.BlockSpec((256, D), lambda i: (i, 0)),
        pl.BlockSpec((256, 1), lambda i: (i, 0)),
        pl.BlockSpec((CD, CD), lambda i: (0, 0)),
    ],
    out_specs=[
        pl.BlockSpec((256, D), lambda i: (i, 0)),
        pl.BlockSpec((256, D), lambda i: (i, 0)),
    ],
    out_shape=[
        jax.ShapeDtypeStruct((NP, D), jnp.float32),
        jax.ShapeDtypeStruct((NP, D), jnp.float32),
    ],
)


_tc_c = pl.pallas_call(
    _tcc_body,
    grid=(NP // 256,),
    in_specs=[
        pl.BlockSpec((256, D), lambda i: (i, 0)),
        pl.BlockSpec((256, D), lambda i: (i, 0)),
        pl.BlockSpec((256, 1), lambda i: (i, 0)),
        pl.BlockSpec((CD, CD), lambda i: (0, 0)),
    ],
    out_specs=[
        pl.BlockSpec((256, D), lambda i: (i, 0)),
        pl.BlockSpec((256, D), lambda i: (i, 0)),
    ],
    out_shape=[
        jax.ShapeDtypeStruct((NP, D), jnp.float32),
        jax.ShapeDtypeStruct((NP, D), jnp.float32),
    ],
)


def _ntn_body(sa_ref, sb_ref, cnt_ref, wmt_ref, wb_ref, bm_ref, wc_ref,
              bc_ref, out_ref):
    cnt1 = jnp.clip(cnt_ref[:, 0:1], 1.0, None)
    cnt2 = jnp.clip(cnt_ref[:, 1:2], 1.0, None)
    g1 = jnp.concatenate([sa_ref[:, 0, :], sb_ref[:, 0, :]], axis=1) / cnt1
    g2 = jnp.concatenate([sa_ref[:, 1, :], sb_ref[:, 1, :]], axis=1) / cnt2
    mb = (jnp.dot(g1, wb_ref[0:CD, :], preferred_element_type=jnp.float32)
          + jnp.dot(g2, wb_ref[CD:2 * CD, :], preferred_element_type=jnp.float32))
    acc = jnp.zeros((G // 2, 2), jnp.float32) + bc_ref[...]
    for t in range(T):
        zt = jnp.dot(g1, wmt_ref[t], preferred_element_type=jnp.float32)
        mst = jnp.sum(zt * g2, axis=1, keepdims=True)
        simt = jnp.tanh(mst + mb[:, t:t + 1] + bm_ref[:, t:t + 1])
        acc = acc + simt * wc_ref[t:t + 1, :]
    out_ref[...] = acc


_tc_ntn = pl.pallas_call(
    _ntn_body,
    out_shape=jax.ShapeDtypeStruct((G // 2, 2), jnp.float32),
)


def kernel(x, edge_index, x_batch, onto_emb, text_emb, Wt, bt, Wg1, Wg2, Wm,
           Wb, bm, Wc, bc):
    x32 = jnp.asarray(x, jnp.int32)
    xp = jnp.concatenate([x32, jnp.zeros((NP - N,), jnp.int32)])
    epad = jnp.full((EP - E,), N, jnp.int32)
    srcp = jnp.concatenate([jnp.asarray(edge_index[0], jnp.int32), epad])
    dstp = jnp.concatenate([jnp.asarray(edge_index[1], jnp.int32), epad])
    xbp = jnp.concatenate([jnp.asarray(x_batch, jnp.int32),
                           jnp.full((NP - N,), G, jnp.int32)])

    dst3 = dstp.reshape(EP // EK, EK)

    onto_x, text_x, deg_p = _encode_call(xp, dst3, onto_emb, text_emb)
    degT = jnp.transpose(deg_p)
    y_a, y_b, dinv2 = _tc_a(onto_x, text_x, degT, Wt, bt.reshape(1, D))

    u1_a, u1_b = _agg_call(y_a, y_b, srcp, dstp)
    y2_a, y2_b = _tc_b(u1_a, u1_b, dinv2, Wg1)
    u2_a, u2_b = _agg_call(y2_a, y2_b, srcp, dstp)
    h2_a, h2_b = _tc_c(u2_a, u2_b, dinv2, Wg2)

    sa, sb, cnt = _pool_call(h2_a, h2_b, xbp)

    sa_p = sa[:G].reshape(G // 2, 2, D)
    sb_p = sb[:G].reshape(G // 2, 2, D)
    cnt_p = cnt[:G, 0].reshape(G // 2, 2)
    WmT = jnp.transpose(Wm, (2, 0, 1))
    logits = _tc_ntn(sa_p, sb_p, cnt_p, WmT, Wb, bm.reshape(1, T), Wc,
                     bc.reshape(1, 2))
    return logits


# speedup vs baseline: 1.5873x; 1.5873x over previous
import functools

import jax
import jax.numpy as jnp
from jax import lax
from jax.experimental import pallas as pl
from jax.experimental.pallas import tpu as pltpu
from jax.experimental.pallas import tpu_sc as plsc

N = 10000
NP = 10240
E = 320000
EP = 327680
D = 128
CD = 2 * D
VOC = 20000
T = 16
G = 256
GPOOL = 384
NC = 2
NS = 16
K = 80
RPT = NP // (NC * NS)
EPC = EP // NC
EPT_DEG = EPC // NS
EPT = EP // NS
NPT = NP // NS
GPT = GPOOL // NS
EK = 128
NSUB = 8
SK = NSUB * EK

_MESH = plsc.VectorSubcoreMesh(core_axis_name="c", subcore_axis_name="s")


def _fill1d(ref, n, val):
    def body(i, _):
        ref[pl.ds(i * 16, 16)] = jnp.full((16,), val, jnp.float32)
        return 0
    lax.fori_loop(0, n // 16, body, 0)


@functools.partial(
    pl.kernel,
    mesh=_MESH,
    out_type=[
        jax.ShapeDtypeStruct((NP, D), jnp.float32),
        jax.ShapeDtypeStruct((NP, 768), jnp.float32),
        jax.ShapeDtypeStruct((NC, NP), jnp.float32),
    ],
    scratch_types=[
        pltpu.VMEM((K,), jnp.int32),
        pltpu.VMEM((NSUB, EK), jnp.int32),
        pltpu.VMEM((K, D), jnp.float32),
        pltpu.VMEM((K, 768), jnp.float32),
        pltpu.VMEM((EK,), jnp.float32),
        pltpu.VMEM((NPT,), jnp.float32),
        pltpu.SemaphoreType.DMA,
        pltpu.SemaphoreType.DMA,
        pltpu.VMEM_SHARED((NP,), jnp.float32),
    ],
)
def _encode_call(x_hbm, dst3_hbm, onto_hbm, text_hbm,
                 onto_out, text_out, deg_out,
                 idx_v, didx_v, onto_v, text_v, ones_v, nbuf_v, sem, sem2,
                 deg_sp):
    c = lax.axis_index("c")
    s = lax.axis_index("s")
    wid = s * NC + c

    _fill1d(ones_v, EK, 1.0)
    _fill1d(nbuf_v, NPT, 0.0)
    pltpu.sync_copy(nbuf_v, deg_sp.at[pl.ds(s * NPT, NPT)])
    plsc.subcore_barrier()

    def gbody(j, _):
        base = wid * RPT + j * K
        pltpu.sync_copy(x_hbm.at[pl.ds(base, K)], idx_v)
        cp1 = pltpu.async_copy(onto_hbm.at[idx_v], onto_v, sem)
        cp2 = pltpu.async_copy(text_hbm.at[idx_v], text_v, sem2)
        cp1.wait()
        co1 = pltpu.async_copy(onto_v, onto_out.at[pl.ds(base, K)], sem)
        cp2.wait()
        co2 = pltpu.async_copy(text_v, text_out.at[pl.ds(base, K)], sem2)
        co1.wait()
        co2.wait()
        return 0
    lax.fori_loop(0, RPT // K, gbody, 0)

    def dbody(j, _):
        rowb = pl.multiple_of((c * EPC + s * EPT_DEG) // EK + j * NSUB, 8)
        pltpu.sync_copy(dst3_hbm.at[pl.ds(rowb, NSUB)], didx_v)
        cps = [pltpu.async_copy(ones_v, deg_sp.at[didx_v.at[jj]], sem,
                                add=True)
               for jj in range(NSUB)]
        for cp in cps:
            cp.wait()
        return 0
    lax.fori_loop(0, EPT_DEG // SK, dbody, 0)
    plsc.subcore_barrier()

    pltpu.sync_copy(deg_sp.at[pl.ds(s * NPT, NPT)], nbuf_v)
    pltpu.sync_copy(nbuf_v, deg_out.at[c, pl.ds(s * NPT, NPT)])


@functools.partial(
    pl.kernel,
    mesh=_MESH,
    out_type=[
        jax.ShapeDtypeStruct((NP, D), jnp.float32),
        jax.ShapeDtypeStruct((NP, D), jnp.float32),
    ],
    scratch_types=[
        pltpu.VMEM((K,), jnp.int32),
        pltpu.VMEM((K, D), jnp.float32),
        pltpu.SemaphoreType.DMA,
        pltpu.VMEM_SHARED((NP, D), jnp.float32),
    ],
)
def _agg_call(ma_hbm, mb_hbm, src_hbm, dst_hbm,
              oa_hbm, ob_hbm,
              idx_v, rows_v, sem, acc_sp):
    c = lax.axis_index("c")
    s = lax.axis_index("s")

    def run(m_hbm, o_hbm):
        def ibody(j, _):
            base = s * NPT + j * K
            pltpu.sync_copy(m_hbm.at[pl.ds(base, K)], rows_v)
            pltpu.sync_copy(rows_v, acc_sp.at[pl.ds(base, K)])
            return 0
        lax.fori_loop(0, NPT // K, ibody, 0)
        plsc.subcore_barrier()

        def ebody(j, _):
            eb = s * EPT + j * K
            pltpu.sync_copy(src_hbm.at[pl.ds(eb, K)], idx_v)
            pltpu.async_copy(m_hbm.at[idx_v], rows_v, sem).wait()
            pltpu.sync_copy(dst_hbm.at[pl.ds(eb, K)], idx_v)
            pltpu.sync_copy(rows_v, acc_sp.at[idx_v], add=True)
            return 0
        lax.fori_loop(0, EPT // K, ebody, 0)
        plsc.subcore_barrier()

        def obody(j, _):
            base = s * NPT + j * K
            pltpu.sync_copy(acc_sp.at[pl.ds(base, K)], rows_v)
            pltpu.sync_copy(rows_v, o_hbm.at[pl.ds(base, K)])
            return 0
        lax.fori_loop(0, NPT // K, obody, 0)

    pl.when(c == 0)(lambda: run(ma_hbm, oa_hbm))
    pl.when(c == 1)(lambda: run(mb_hbm, ob_hbm))


@functools.partial(
    pl.kernel,
    mesh=_MESH,
    out_type=[
        jax.ShapeDtypeStruct((GPOOL, D), jnp.float32),
        jax.ShapeDtypeStruct((GPOOL, D), jnp.float32),
        jax.ShapeDtypeStruct((GPOOL, D), jnp.float32),
    ],
    scratch_types=[
        pltpu.VMEM((K,), jnp.int32),
        pltpu.VMEM((K, D), jnp.float32),
        pltpu.VMEM((K, D), jnp.float32),
        pltpu.SemaphoreType.DMA,
        pltpu.VMEM_SHARED((GPOOL, D), jnp.float32),
        pltpu.VMEM_SHARED((GPOOL, D), jnp.float32),
    ],
)
def _pool_call(ha_hbm, hb_hbm, xb_hbm,
               sa_hbm, sb_hbm, cnt_hbm,
               idx_v, buf_v, ones_v, sem, pool_sp, cnt_sp):
    c = lax.axis_index("c")
    s = lax.axis_index("s")

    def zrow(i, _):
        for q in range(D // 16):
            buf_v[i, pl.ds(q * 16, 16)] = jnp.zeros((16,), jnp.float32)
            ones_v[i, pl.ds(q * 16, 16)] = jnp.full((16,), 1.0, jnp.float32)
        return 0
    lax.fori_loop(0, K, zrow, 0)

    pltpu.sync_copy(buf_v.at[pl.ds(0, GPT)], pool_sp.at[pl.ds(s * GPT, GPT)])
    pltpu.sync_copy(buf_v.at[pl.ds(0, GPT)], cnt_sp.at[pl.ds(s * GPT, GPT)])
    plsc.subcore_barrier()

    def run(h_hbm):
        def pbody(j, _):
            base = s * NPT + j * K
            pltpu.sync_copy(xb_hbm.at[pl.ds(base, K)], idx_v)
            pltpu.sync_copy(h_hbm.at[pl.ds(base, K)], buf_v)
            pltpu.sync_copy(buf_v, pool_sp.at[idx_v], add=True)

            @pl.when(c == 0)
            def _():
                pltpu.sync_copy(ones_v, cnt_sp.at[idx_v], add=True)
            return 0
        lax.fori_loop(0, NPT // K, pbody, 0)

    pl.when(c == 0)(lambda: run(ha_hbm))
    pl.when(c == 1)(lambda: run(hb_hbm))
    plsc.subcore_barrier()

    pltpu.sync_copy(pool_sp.at[pl.ds(s * GPT, GPT)], buf_v.at[pl.ds(0, GPT)])

    @pl.when(c == 0)
    def _():
        pltpu.sync_copy(buf_v.at[pl.ds(0, GPT)], sa_hbm.at[pl.ds(s * GPT, GPT)])
        pltpu.sync_copy(cnt_sp.at[pl.ds(s * GPT, GPT)], ones_v.at[pl.ds(0, GPT)])
        pltpu.sync_copy(ones_v.at[pl.ds(0, GPT)], cnt_hbm.at[pl.ds(s * GPT, GPT)])

    @pl.when(c == 1)
    def _():
        pltpu.sync_copy(buf_v.at[pl.ds(0, GPT)], sb_hbm.at[pl.ds(s * GPT, GPT)])


def _tca_body(onto_ref, text_ref, degT_ref, wt_ref, bt_ref,
              ma_ref, mb_ref, dv_ref):
    dv = lax.rsqrt(degT_ref[:, 0:1] + degT_ref[:, 1:2] + 1.0)
    t = jnp.dot(text_ref[...], wt_ref[...],
                preferred_element_type=jnp.float32) + bt_ref[...]
    ma_ref[...] = onto_ref[...] * dv
    mb_ref[...] = t * dv
    dv_ref[...] = dv


_tc_a = pl.pallas_call(
    _tca_body,
    grid=(NP // 256,),
    in_specs=[
        pl.BlockSpec((256, D), lambda i: (i, 0)),
        pl.BlockSpec((256, 768), lambda i: (i, 0)),
        pl.BlockSpec((256, NC), lambda i: (i, 0)),
        pl.BlockSpec((768, D), lambda i: (0, 0)),
        pl.BlockSpec((1, D), lambda i: (0, 0)),
    ],
    out_specs=[
        pl.BlockSpec((256, D), lambda i: (i, 0)),
        pl.BlockSpec((256, D), lambda i: (i, 0)),
        pl.BlockSpec((256, 1), lambda i: (i, 0)),
    ],
    out_shape=[
        jax.ShapeDtypeStruct((NP, D), jnp.float32),
        jax.ShapeDtypeStruct((NP, D), jnp.float32),
        jax.ShapeDtypeStruct((NP, 1), jnp.float32),
    ],
)


def _tcb_body(aa_ref, ab_ref, dv_ref, wg1_ref, m2a_ref, m2b_ref):
    dv = dv_ref[...]
    u = jnp.concatenate([aa_ref[...], ab_ref[...]], axis=1) * dv
    h1 = jnp.maximum(jnp.dot(u, wg1_ref[...],
                             preferred_element_type=jnp.float32), 0.0)
    m2 = h1 * dv
    m2a_ref[...] = m2[:, 0:D]
    m2b_ref[...] = m2[:, D:CD]


def _tcc_body(aa_ref, ab_ref, dv_ref, wg2_ref, h2a_ref, h2b_ref):
    u = jnp.concatenate([aa_ref[...], ab_ref[...]], axis=1) * dv_ref[...]
    h2 = jnp.dot(u, wg2_ref[...], preferred_element_type=jnp.float32)
    h2a_ref[...] = h2[:, 0:D]
    h2b_ref[...] = h2[:, D:CD]


_tc_b = pl.pallas_call(
    _tcb_body,
    grid=(NP // 256,),
    in_specs=[
        pl.BlockSpec((256, D), lambda i: (i, 0)),
        pl.BlockSpec((256, D), lambda i: (i, 0)),
        pl.BlockSpec((256, 1), lambda i: (i, 0)),
        pl.BlockSpec((CD, CD), lambda i: (0, 0)),
    ],
    out_specs=[
        pl.BlockSpec((256, D), lambda i: (i, 0)),
        pl.BlockSpec((256, D), lambda i: (i, 0)),
    ],
    out_shape=[
        jax.ShapeDtypeStruct((NP, D), jnp.float32),
        jax.ShapeDtypeStruct((NP, D), jnp.float32),
    ],
)


_tc_c = pl.pallas_call(
    _tcc_body,
    grid=(NP // 256,),
    in_specs=[
        pl.BlockSpec((256, D), lambda i: (i, 0)),
        pl.BlockSpec((256, D), lambda i: (i, 0)),
        pl.BlockSpec((256, 1), lambda i: (i, 0)),
        pl.BlockSpec((CD, CD), lambda i: (0, 0)),
    ],
    out_specs=[
        pl.BlockSpec((256, D), lambda i: (i, 0)),
        pl.BlockSpec((256, D), lambda i: (i, 0)),
    ],
    out_shape=[
        jax.ShapeDtypeStruct((NP, D), jnp.float32),
        jax.ShapeDtypeStruct((NP, D), jnp.float32),
    ],
)


def _ntn_body(sa_ref, sb_ref, cnt_ref, wmt_ref, wb_ref, bm_ref, wc_ref,
              bc_ref, out_ref):
    cnt1 = jnp.clip(cnt_ref[:, 0:1], 1.0, None)
    cnt2 = jnp.clip(cnt_ref[:, 1:2], 1.0, None)
    g1 = jnp.concatenate([sa_ref[:, 0, :], sb_ref[:, 0, :]], axis=1) / cnt1
    g2 = jnp.concatenate([sa_ref[:, 1, :], sb_ref[:, 1, :]], axis=1) / cnt2
    mb = (jnp.dot(g1, wb_ref[0:CD, :], preferred_element_type=jnp.float32)
          + jnp.dot(g2, wb_ref[CD:2 * CD, :], preferred_element_type=jnp.float32))
    acc = jnp.zeros((G // 2, 2), jnp.float32) + bc_ref[...]
    for t in range(T):
        zt = jnp.dot(g1, wmt_ref[t], preferred_element_type=jnp.float32)
        mst = jnp.sum(zt * g2, axis=1, keepdims=True)
        simt = jnp.tanh(mst + mb[:, t:t + 1] + bm_ref[:, t:t + 1])
        acc = acc + simt * wc_ref[t:t + 1, :]
    out_ref[...] = acc


_tc_ntn = pl.pallas_call(
    _ntn_body,
    out_shape=jax.ShapeDtypeStruct((G // 2, 2), jnp.float32),
)


def kernel(x, edge_index, x_batch, onto_emb, text_emb, Wt, bt, Wg1, Wg2, Wm,
           Wb, bm, Wc, bc):
    x32 = jnp.asarray(x, jnp.int32)
    xp = jnp.concatenate([x32, jnp.zeros((NP - N,), jnp.int32)])
    epad = N + jnp.arange(EP - E, dtype=jnp.int32) % (NP - N)
    srcp = jnp.concatenate([jnp.asarray(edge_index[0], jnp.int32), epad])
    dstp = jnp.concatenate([jnp.asarray(edge_index[1], jnp.int32), epad])
    xbp = jnp.concatenate([jnp.asarray(x_batch, jnp.int32),
                           jnp.full((NP - N,), G, jnp.int32)])

    dst3 = dstp.reshape(EP // EK, EK)

    onto_x, text_x, deg_p = _encode_call(xp, dst3, onto_emb, text_emb)
    degT = jnp.transpose(deg_p)
    y_a, y_b, dinv2 = _tc_a(onto_x, text_x, degT, Wt, bt.reshape(1, D))

    u1_a, u1_b = _agg_call(y_a, y_b, srcp, dstp)
    y2_a, y2_b = _tc_b(u1_a, u1_b, dinv2, Wg1)
    u2_a, u2_b = _agg_call(y2_a, y2_b, srcp, dstp)
    h2_a, h2_b = _tc_c(u2_a, u2_b, dinv2, Wg2)

    sa, sb, cnt = _pool_call(h2_a, h2_b, xbp)

    sa_p = sa[:G].reshape(G // 2, 2, D)
    sb_p = sb[:G].reshape(G // 2, 2, D)
    cnt_p = cnt[:G, 0].reshape(G // 2, 2)
    WmT = jnp.transpose(Wm, (2, 0, 1))
    logits = _tc_ntn(sa_p, sb_p, cnt_p, WmT, Wb, bm.reshape(1, T), Wc,
                     bc.reshape(1, 2))
    return logits


# speedup vs baseline: 2.2345x; 1.4077x over previous
import functools

import jax
import jax.numpy as jnp
from jax import lax
from jax.experimental import pallas as pl
from jax.experimental.pallas import tpu as pltpu
from jax.experimental.pallas import tpu_sc as plsc

N = 10000
NP = 10240
E = 320000
EP = 327680
D = 128
CD = 2 * D
VOC = 20000
T = 16
G = 256
GPOOL = 384
NC = 2
NS = 16
K = 80
RPT = NP // (NC * NS)
EPC = EP // NC
EPT_DEG = EPC // NS
EPT = EP // NS
NPT = NP // NS
GPT = GPOOL // NS
EK = 128
NSUB = 8
SK = NSUB * EK

_MESH = plsc.VectorSubcoreMesh(core_axis_name="c", subcore_axis_name="s")


def _fill1d(ref, n, val):
    def body(i, _):
        ref[pl.ds(i * 16, 16)] = jnp.full((16,), val, jnp.float32)
        return 0
    lax.fori_loop(0, n // 16, body, 0)


@functools.partial(
    pl.kernel,
    mesh=_MESH,
    out_type=[
        jax.ShapeDtypeStruct((NP, D), jnp.float32),
        jax.ShapeDtypeStruct((NP, 768), jnp.float32),
        jax.ShapeDtypeStruct((NC, NP), jnp.float32),
    ],
    scratch_types=[
        pltpu.VMEM((K,), jnp.int32),
        pltpu.VMEM((NSUB, EK), jnp.int32),
        pltpu.VMEM((K, D), jnp.float32),
        pltpu.VMEM((K, 768), jnp.float32),
        pltpu.VMEM((EK,), jnp.float32),
        pltpu.VMEM((NPT,), jnp.float32),
        pltpu.SemaphoreType.DMA,
        pltpu.SemaphoreType.DMA,
        pltpu.VMEM_SHARED((NP,), jnp.float32),
    ],
)
def _encode_call(x_hbm, dst3_hbm, onto_hbm, text_hbm,
                 onto_out, text_out, deg_out,
                 idx_v, didx_v, onto_v, text_v, ones_v, nbuf_v, sem, sem2,
                 deg_sp):
    c = lax.axis_index("c")
    s = lax.axis_index("s")
    wid = s * NC + c

    _fill1d(ones_v, EK, 1.0)
    _fill1d(nbuf_v, NPT, 0.0)
    pltpu.sync_copy(nbuf_v, deg_sp.at[pl.ds(s * NPT, NPT)])
    plsc.subcore_barrier()

    def gbody(j, _):
        base = wid * RPT + j * K
        pltpu.sync_copy(x_hbm.at[pl.ds(base, K)], idx_v)
        cp1 = pltpu.async_copy(onto_hbm.at[idx_v], onto_v, sem)
        cp2 = pltpu.async_copy(text_hbm.at[idx_v], text_v, sem2)
        cp1.wait()
        co1 = pltpu.async_copy(onto_v, onto_out.at[pl.ds(base, K)], sem)
        cp2.wait()
        co2 = pltpu.async_copy(text_v, text_out.at[pl.ds(base, K)], sem2)
        co1.wait()
        co2.wait()
        return 0
    lax.fori_loop(0, RPT // K, gbody, 0)

    def dbody(j, _):
        rowb = pl.multiple_of((c * EPC + s * EPT_DEG) // EK + j * NSUB, 8)
        pltpu.sync_copy(dst3_hbm.at[pl.ds(rowb, NSUB)], didx_v)
        cps = [pltpu.async_copy(ones_v, deg_sp.at[didx_v.at[jj]], sem,
                                add=True)
               for jj in range(NSUB)]
        for cp in cps:
            cp.wait()
        return 0
    lax.fori_loop(0, EPT_DEG // SK, dbody, 0)
    plsc.subcore_barrier()

    pltpu.sync_copy(deg_sp.at[pl.ds(s * NPT, NPT)], nbuf_v)
    pltpu.sync_copy(nbuf_v, deg_out.at[c, pl.ds(s * NPT, NPT)])


@functools.partial(
    pl.kernel,
    mesh=_MESH,
    out_type=[
        jax.ShapeDtypeStruct((NP, D), jnp.float32),
        jax.ShapeDtypeStruct((NP, D), jnp.float32),
    ],
    scratch_types=[
        pltpu.VMEM((K,), jnp.int32),
        pltpu.VMEM((K,), jnp.int32),
        pltpu.VMEM((K,), jnp.int32),
        pltpu.VMEM((K,), jnp.int32),
        pltpu.VMEM((K, D), jnp.float32),
        pltpu.VMEM((K, D), jnp.float32),
        pltpu.SemaphoreType.DMA,
        pltpu.SemaphoreType.DMA,
        pltpu.VMEM_SHARED((NP, D), jnp.float32),
    ],
)
def _agg_call(ma_hbm, mb_hbm, src_hbm, dst_hbm,
              oa_hbm, ob_hbm,
              ia_v, da_v, ib_v, db_v, rows_v, rowsb_v, gsem, ssem, acc_sp):
    idx_v = ia_v
    c = lax.axis_index("c")
    s = lax.axis_index("s")

    def run(m_hbm, o_hbm):
        def ibody(j, _):
            base = s * NPT + j * K
            pltpu.sync_copy(m_hbm.at[pl.ds(base, K)], rows_v)
            pltpu.sync_copy(rows_v, acc_sp.at[pl.ds(base, K)])
            return 0
        lax.fori_loop(0, NPT // K, ibody, 0)
        plsc.subcore_barrier()

        def ebody(j, _):
            eba = s * EPT + j * (2 * K)
            ebb = eba + K
            pltpu.sync_copy(src_hbm.at[pl.ds(eba, K)], ia_v)
            ga = pltpu.async_copy(m_hbm.at[ia_v], rows_v, gsem)
            pltpu.sync_copy(dst_hbm.at[pl.ds(eba, K)], da_v)
            pltpu.sync_copy(src_hbm.at[pl.ds(ebb, K)], ib_v)
            ga.wait()
            gb = pltpu.async_copy(m_hbm.at[ib_v], rowsb_v, gsem)
            sa = pltpu.async_copy(rows_v, acc_sp.at[da_v], ssem, add=True)
            pltpu.sync_copy(dst_hbm.at[pl.ds(ebb, K)], db_v)
            gb.wait()
            sa.wait()
            pltpu.sync_copy(rowsb_v, acc_sp.at[db_v], add=True)
            return 0
        lax.fori_loop(0, EPT // (2 * K), ebody, 0)
        plsc.subcore_barrier()

        def obody(j, _):
            base = s * NPT + j * K
            pltpu.sync_copy(acc_sp.at[pl.ds(base, K)], rows_v)
            pltpu.sync_copy(rows_v, o_hbm.at[pl.ds(base, K)])
            return 0
        lax.fori_loop(0, NPT // K, obody, 0)

    pl.when(c == 0)(lambda: run(ma_hbm, oa_hbm))
    pl.when(c == 1)(lambda: run(mb_hbm, ob_hbm))


@functools.partial(
    pl.kernel,
    mesh=_MESH,
    out_type=[
        jax.ShapeDtypeStruct((GPOOL, D), jnp.float32),
        jax.ShapeDtypeStruct((GPOOL, D), jnp.float32),
        jax.ShapeDtypeStruct((GPOOL, D), jnp.float32),
    ],
    scratch_types=[
        pltpu.VMEM((K,), jnp.int32),
        pltpu.VMEM((K, D), jnp.float32),
        pltpu.VMEM((K, D), jnp.float32),
        pltpu.SemaphoreType.DMA,
        pltpu.VMEM_SHARED((GPOOL, D), jnp.float32),
        pltpu.VMEM_SHARED((GPOOL, D), jnp.float32),
    ],
)
def _pool_call(ha_hbm, hb_hbm, xb_hbm,
               sa_hbm, sb_hbm, cnt_hbm,
               idx_v, buf_v, ones_v, sem, pool_sp, cnt_sp):
    c = lax.axis_index("c")
    s = lax.axis_index("s")

    def zrow(i, _):
        for q in range(D // 16):
            buf_v[i, pl.ds(q * 16, 16)] = jnp.zeros((16,), jnp.float32)
            ones_v[i, pl.ds(q * 16, 16)] = jnp.full((16,), 1.0, jnp.float32)
        return 0
    lax.fori_loop(0, K, zrow, 0)

    pltpu.sync_copy(buf_v.at[pl.ds(0, GPT)], pool_sp.at[pl.ds(s * GPT, GPT)])
    pltpu.sync_copy(buf_v.at[pl.ds(0, GPT)], cnt_sp.at[pl.ds(s * GPT, GPT)])
    plsc.subcore_barrier()

    def run(h_hbm):
        def pbody(j, _):
            base = s * NPT + j * K
            pltpu.sync_copy(xb_hbm.at[pl.ds(base, K)], idx_v)
            pltpu.sync_copy(h_hbm.at[pl.ds(base, K)], buf_v)
            pltpu.sync_copy(buf_v, pool_sp.at[idx_v], add=True)

            @pl.when(c == 0)
            def _():
                pltpu.sync_copy(ones_v, cnt_sp.at[idx_v], add=True)
            return 0
        lax.fori_loop(0, NPT // K, pbody, 0)

    pl.when(c == 0)(lambda: run(ha_hbm))
    pl.when(c == 1)(lambda: run(hb_hbm))
    plsc.subcore_barrier()

    pltpu.sync_copy(pool_sp.at[pl.ds(s * GPT, GPT)], buf_v.at[pl.ds(0, GPT)])

    @pl.when(c == 0)
    def _():
        pltpu.sync_copy(buf_v.at[pl.ds(0, GPT)], sa_hbm.at[pl.ds(s * GPT, GPT)])
        pltpu.sync_copy(cnt_sp.at[pl.ds(s * GPT, GPT)], ones_v.at[pl.ds(0, GPT)])
        pltpu.sync_copy(ones_v.at[pl.ds(0, GPT)], cnt_hbm.at[pl.ds(s * GPT, GPT)])

    @pl.when(c == 1)
    def _():
        pltpu.sync_copy(buf_v.at[pl.ds(0, GPT)], sb_hbm.at[pl.ds(s * GPT, GPT)])


def _tca_body(onto_ref, text_ref, degT_ref, wt_ref, bt_ref,
              ma_ref, mb_ref, dv_ref):
    dv = lax.rsqrt(degT_ref[:, 0:1] + degT_ref[:, 1:2] + 1.0)
    t = jnp.dot(text_ref[...], wt_ref[...],
                preferred_element_type=jnp.float32) + bt_ref[...]
    ma_ref[...] = onto_ref[...] * dv
    mb_ref[...] = t * dv
    dv_ref[...] = dv


_tc_a = pl.pallas_call(
    _tca_body,
    grid=(NP // 256,),
    in_specs=[
        pl.BlockSpec((256, D), lambda i: (i, 0)),
        pl.BlockSpec((256, 768), lambda i: (i, 0)),
        pl.BlockSpec((256, NC), lambda i: (i, 0)),
        pl.BlockSpec((768, D), lambda i: (0, 0)),
        pl.BlockSpec((1, D), lambda i: (0, 0)),
    ],
    out_specs=[
        pl.BlockSpec((256, D), lambda i: (i, 0)),
        pl.BlockSpec((256, D), lambda i: (i, 0)),
        pl.BlockSpec((256, 1), lambda i: (i, 0)),
    ],
    out_shape=[
        jax.ShapeDtypeStruct((NP, D), jnp.float32),
        jax.ShapeDtypeStruct((NP, D), jnp.float32),
        jax.ShapeDtypeStruct((NP, 1), jnp.float32),
    ],
)


def _tcb_body(aa_ref, ab_ref, dv_ref, wg1_ref, m2a_ref, m2b_ref):
    dv = dv_ref[...]
    u = jnp.concatenate([aa_ref[...], ab_ref[...]], axis=1) * dv
    h1 = jnp.maximum(jnp.dot(u, wg1_ref[...],
                             preferred_element_type=jnp.float32), 0.0)
    m2 = h1 * dv
    m2a_ref[...] = m2[:, 0:D]
    m2b_ref[...] = m2[:, D:CD]


def _tcc_body(aa_ref, ab_ref, dv_ref, wg2_ref, h2a_ref, h2b_ref):
    u = jnp.concatenate([aa_ref[...], ab_ref[...]], axis=1) * dv_ref[...]
    h2 = jnp.dot(u, wg2_ref[...], preferred_element_type=jnp.float32)
    h2a_ref[...] = h2[:, 0:D]
    h2b_ref[...] = h2[:, D:CD]


_tc_b = pl.pallas_call(
    _tcb_body,
    grid=(NP // 256,),
    in_specs=[
        pl.BlockSpec((256, D), lambda i: (i, 0)),
        pl.BlockSpec((256, D), lambda i: (i, 0)),
        pl.BlockSpec((256, 1), lambda i: (i, 0)),
        pl.BlockSpec((CD, CD), lambda i: (0, 0)),
    ],
    out_specs=[
        pl.BlockSpec((256, D), lambda i: (i, 0)),
        pl.BlockSpec((256, D), lambda i: (i, 0)),
    ],
    out_shape=[
        jax.ShapeDtypeStruct((NP, D), jnp.float32),
        jax.ShapeDtypeStruct((NP, D), jnp.float32),
    ],
)


_tc_c = pl.pallas_call(
    _tcc_body,
    grid=(NP // 256,),
    in_specs=[
        pl.BlockSpec((256, D), lambda i: (i, 0)),
        pl.BlockSpec((256, D), lambda i: (i, 0)),
        pl.BlockSpec((256, 1), lambda i: (i, 0)),
        pl.BlockSpec((CD, CD), lambda i: (0, 0)),
    ],
    out_specs=[
        pl.BlockSpec((256, D), lambda i: (i, 0)),
        pl.BlockSpec((256, D), lambda i: (i, 0)),
    ],
    out_shape=[
        jax.ShapeDtypeStruct((NP, D), jnp.float32),
        jax.ShapeDtypeStruct((NP, D), jnp.float32),
    ],
)


def _ntn_body(sa_ref, sb_ref, cnt_ref, wmt_ref, wb_ref, bm_ref, wc_ref,
              bc_ref, out_ref):
    cnt1 = jnp.clip(cnt_ref[:, 0:1], 1.0, None)
    cnt2 = jnp.clip(cnt_ref[:, 1:2], 1.0, None)
    g1 = jnp.concatenate([sa_ref[:, 0, :], sb_ref[:, 0, :]], axis=1) / cnt1
    g2 = jnp.concatenate([sa_ref[:, 1, :], sb_ref[:, 1, :]], axis=1) / cnt2
    mb = (jnp.dot(g1, wb_ref[0:CD, :], preferred_element_type=jnp.float32)
          + jnp.dot(g2, wb_ref[CD:2 * CD, :], preferred_element_type=jnp.float32))
    acc = jnp.zeros((G // 2, 2), jnp.float32) + bc_ref[...]
    for t in range(T):
        zt = jnp.dot(g1, wmt_ref[t], preferred_element_type=jnp.float32)
        mst = jnp.sum(zt * g2, axis=1, keepdims=True)
        simt = jnp.tanh(mst + mb[:, t:t + 1] + bm_ref[:, t:t + 1])
        acc = acc + simt * wc_ref[t:t + 1, :]
    out_ref[...] = acc


_tc_ntn = pl.pallas_call(
    _ntn_body,
    out_shape=jax.ShapeDtypeStruct((G // 2, 2), jnp.float32),
)


def kernel(x, edge_index, x_batch, onto_emb, text_emb, Wt, bt, Wg1, Wg2, Wm,
           Wb, bm, Wc, bc):
    x32 = jnp.asarray(x, jnp.int32)
    xp = jnp.concatenate([x32, jnp.zeros((NP - N,), jnp.int32)])
    epad = N + jnp.arange(EP - E, dtype=jnp.int32) % (NP - N)
    srcp = jnp.concatenate([jnp.asarray(edge_index[0], jnp.int32), epad])
    dstp = jnp.concatenate([jnp.asarray(edge_index[1], jnp.int32), epad])
    xbp = jnp.concatenate([jnp.asarray(x_batch, jnp.int32),
                           jnp.full((NP - N,), G, jnp.int32)])

    dst3 = dstp.reshape(EP // EK, EK)

    onto_x, text_x, deg_p = _encode_call(xp, dst3, onto_emb, text_emb)
    degT = jnp.transpose(deg_p)
    y_a, y_b, dinv2 = _tc_a(onto_x, text_x, degT, Wt, bt.reshape(1, D))

    u1_a, u1_b = _agg_call(y_a, y_b, srcp, dstp)
    y2_a, y2_b = _tc_b(u1_a, u1_b, dinv2, Wg1)
    u2_a, u2_b = _agg_call(y2_a, y2_b, srcp, dstp)
    h2_a, h2_b = _tc_c(u2_a, u2_b, dinv2, Wg2)

    sa, sb, cnt = _pool_call(h2_a, h2_b, xbp)

    sa_p = sa[:G].reshape(G // 2, 2, D)
    sb_p = sb[:G].reshape(G // 2, 2, D)
    cnt_p = cnt[:G, 0].reshape(G // 2, 2)
    WmT = jnp.transpose(Wm, (2, 0, 1))
    logits = _tc_ntn(sa_p, sb_p, cnt_p, WmT, Wb, bm.reshape(1, T), Wc,
                     bc.reshape(1, 2))
    return logits
